# baseline (speedup 1.0000x reference)
import jax
import jax.numpy as jnp
from jax.experimental import pallas as pl


def _pick_block(t: int, target: int = 512) -> int:
    for cand in (target, 256, 128, 64, 32, 16, 8):
        if t % cand == 0 and cand <= t:
            return cand
    return t


def kernel(series, W1, b1, W2, b2, codebook, g_noise):
    B, F, T = series.shape
    H = W1.shape[0]
    C = W2.shape[0]
    G = F // codebook.shape[2]
    CG = C // G
    FG = F // G

    blk = _pick_block(T)

    gn3 = g_noise.reshape(B, T * G, CG)
    cbT = codebook.reshape(G, CG, FG).transpose(0, 2, 1).reshape(G * FG, CG)
    b1c = b1.reshape(H, 1)
    b2c = b2.reshape(C, 1)
    r_io = jnp.arange(G * blk)
    sel = (r_io[:, None] == (G * (r_io[None, :] % blk) + r_io[None, :] // blk)
           ).astype(jnp.float32)

    def kern(s_ref, w1_ref, b1_ref, w2_ref, b2_ref, cb_ref, gn_ref, sel_ref,
             q_ref, idx_ref):
        s = s_ref[0]
        h = jnp.dot(w1_ref[...], s, preferred_element_type=jnp.float32)
        h = jnp.maximum(h + b1_ref[...], 0.0)
        z = jnp.dot(w2_ref[...], h, preferred_element_type=jnp.float32)
        zb = z + b2_ref[...]
        nT = jax.lax.dot_general(
            gn_ref[0], sel_ref[...], (((0,), (0,)), ((), ())),
            precision=jax.lax.Precision.HIGHEST,
            preferred_element_type=jnp.float32)
        idx_rows = []
        for g in range(G):
            l = zb[g * CG:(g + 1) * CG, :] + nT[:, g * blk:(g + 1) * blk]
            m = jnp.max(l, axis=0, keepdims=True)
            iota = jax.lax.broadcasted_iota(jnp.int32, (CG, blk), 0)
            idxg = jnp.min(jnp.where(l == m, iota, CG), axis=0, keepdims=True)
            oh = (iota == idxg).astype(jnp.float32)
            q_ref[0, g * FG:(g + 1) * FG, :] = jnp.dot(
                cb_ref[g * FG:(g + 1) * FG, :], oh,
                preferred_element_type=jnp.float32)
            idx_rows.append(idxg)
        idx_ref[0] = jnp.concatenate(idx_rows, axis=0)

    q, idx = pl.pallas_call(
        kern,
        grid=(B, T // blk),
        in_specs=[
            pl.BlockSpec((1, F, blk), lambda b, t: (b, 0, t)),
            pl.BlockSpec((H, F), lambda b, t: (0, 0)),
            pl.BlockSpec((H, 1), lambda b, t: (0, 0)),
            pl.BlockSpec((C, H), lambda b, t: (0, 0)),
            pl.BlockSpec((C, 1), lambda b, t: (0, 0)),
            pl.BlockSpec((G * FG, CG), lambda b, t: (0, 0)),
            pl.BlockSpec((1, G * blk, CG), lambda b, t: (b, t, 0)),
            pl.BlockSpec((G * blk, G * blk), lambda b, t: (0, 0)),
        ],
        out_specs=[
            pl.BlockSpec((1, F, blk), lambda b, t: (b, 0, t)),
            pl.BlockSpec((1, G, blk), lambda b, t: (b, 0, t)),
        ],
        out_shape=[
            jax.ShapeDtypeStruct((B, F, T), jnp.float32),
            jax.ShapeDtypeStruct((B, G, T), jnp.int32),
        ],
    )(series, W1, b1c, W2, b2c, cbT, gn3, sel)
    return q, idx.transpose(0, 2, 1)


# speedup vs baseline: 1.3720x; 1.3720x over previous
import jax
import jax.numpy as jnp
from jax.experimental import pallas as pl


def _pick_block(t: int, target: int = 512) -> int:
    for cand in (target, 256, 128, 64, 32, 16, 8):
        if t % cand == 0 and cand <= t:
            return cand
    return t


def kernel(series, W1, b1, W2, b2, codebook, g_noise):
    B, F, T = series.shape
    H = W1.shape[0]
    C = W2.shape[0]
    G = F // codebook.shape[2]
    CG = C // G
    FG = F // G

    blk = _pick_block(T)

    gn3 = g_noise.reshape(B, T * G, CG)
    cbT = codebook.reshape(G, CG, FG).transpose(0, 2, 1).reshape(G * FG, CG)
    b1c = b1.reshape(H, 1)
    b2c = b2.reshape(C, 1)
    r_io = jnp.arange(G * blk)
    sel = (r_io[:, None] == (G * (r_io[None, :] % blk) + r_io[None, :] // blk)
           ).astype(jnp.bfloat16)

    def kern(s_ref, w1_ref, b1_ref, w2_ref, b2_ref, cb_ref, gn_ref, sel_ref,
             q_ref, idx_ref):
        s = s_ref[0]
        h = jnp.dot(w1_ref[...], s, preferred_element_type=jnp.float32)
        h = jnp.maximum(h + b1_ref[...], 0.0)
        z = jnp.dot(w2_ref[...], h, preferred_element_type=jnp.float32)
        zb = z + b2_ref[...]
        gn = gn_ref[0]
        n_hi = gn.astype(jnp.bfloat16)
        res1 = gn - n_hi.astype(jnp.float32)
        n_mid = res1.astype(jnp.bfloat16)
        n_lo = (res1 - n_mid.astype(jnp.float32)).astype(jnp.bfloat16)
        dims = (((0,), (0,)), ((), ()))
        selv = sel_ref[...]
        nT = (jax.lax.dot_general(n_hi, selv, dims,
                                  preferred_element_type=jnp.float32)
              + jax.lax.dot_general(n_mid, selv, dims,
                                    preferred_element_type=jnp.float32)
              + jax.lax.dot_general(n_lo, selv, dims,
                                    preferred_element_type=jnp.float32))
        idx_rows = []
        for g in range(G):
            l = zb[g * CG:(g + 1) * CG, :] + nT[:, g * blk:(g + 1) * blk]
            m = jnp.max(l, axis=0, keepdims=True)
            iota = jax.lax.broadcasted_iota(jnp.int32, (CG, blk), 0)
            idxg = jnp.min(jnp.where(l == m, iota, CG), axis=0, keepdims=True)
            oh = (iota == idxg).astype(jnp.float32)
            q_ref[0, g * FG:(g + 1) * FG, :] = jnp.dot(
                cb_ref[g * FG:(g + 1) * FG, :], oh,
                preferred_element_type=jnp.float32)
            idx_rows.append(idxg)
        idx_ref[0] = jnp.concatenate(idx_rows, axis=0)

    q, idx = pl.pallas_call(
        kern,
        grid=(B, T // blk),
        in_specs=[
            pl.BlockSpec((1, F, blk), lambda b, t: (b, 0, t)),
            pl.BlockSpec((H, F), lambda b, t: (0, 0)),
            pl.BlockSpec((H, 1), lambda b, t: (0, 0)),
            pl.BlockSpec((C, H), lambda b, t: (0, 0)),
            pl.BlockSpec((C, 1), lambda b, t: (0, 0)),
            pl.BlockSpec((G * FG, CG), lambda b, t: (0, 0)),
            pl.BlockSpec((1, G * blk, CG), lambda b, t: (b, t, 0)),
            pl.BlockSpec((G * blk, G * blk), lambda b, t: (0, 0)),
        ],
        out_specs=[
            pl.BlockSpec((1, F, blk), lambda b, t: (b, 0, t)),
            pl.BlockSpec((1, G, blk), lambda b, t: (b, 0, t)),
        ],
        out_shape=[
            jax.ShapeDtypeStruct((B, F, T), jnp.float32),
            jax.ShapeDtypeStruct((B, G, T), jnp.int32),
        ],
    )(series, W1, b1c, W2, b2c, cbT, gn3, sel)
    return q, idx.transpose(0, 2, 1)


# speedup vs baseline: 2.0138x; 1.4677x over previous
import jax
import jax.numpy as jnp
from jax.experimental import pallas as pl


def _pick_block(t: int, target: int = 512) -> int:
    for cand in (target, 256, 128, 64, 32, 16, 8):
        if t % cand == 0 and cand <= t:
            return cand
    return t


def kernel(series, W1, b1, W2, b2, codebook, g_noise):
    B, F, T = series.shape
    H = W1.shape[0]
    C = W2.shape[0]
    G = F // codebook.shape[2]
    CG = C // G
    FG = F // G

    blk = _pick_block(T)

    gn3 = g_noise.reshape(B, T * G, CG)
    cbT = codebook.reshape(G, CG, FG).transpose(0, 2, 1).reshape(G * FG, CG)
    b1c = b1.reshape(H, 1)
    b2c = b2.reshape(C, 1)

    def kern(s_ref, w1_ref, b1_ref, w2_ref, b2_ref, cb_ref, gn_ref,
             q_ref, idx_ref):
        s = s_ref[0]
        h = jnp.dot(w1_ref[...], s, preferred_element_type=jnp.float32)
        h = jnp.maximum(h + b1_ref[...], 0.0)
        z = jnp.dot(w2_ref[...], h, preferred_element_type=jnp.float32)
        zb = z + b2_ref[...]
        gn_d = jnp.transpose(gn_ref[0].reshape(blk, G, CG), (1, 0, 2))
        idx_rows = []
        for g in range(G):
            l = zb[g * CG:(g + 1) * CG, :] + gn_d[g].T
            m = jnp.max(l, axis=0, keepdims=True)
            iota = jax.lax.broadcasted_iota(jnp.int32, (CG, blk), 0)
            idxg = jnp.min(jnp.where(l == m, iota, CG), axis=0, keepdims=True)
            oh = (iota == idxg).astype(jnp.float32)
            q_ref[0, g * FG:(g + 1) * FG, :] = jnp.dot(
                cb_ref[g * FG:(g + 1) * FG, :], oh,
                preferred_element_type=jnp.float32)
            idx_rows.append(idxg)
        idx_ref[0] = jnp.concatenate(idx_rows, axis=0)

    q, idx = pl.pallas_call(
        kern,
        grid=(B, T // blk),
        in_specs=[
            pl.BlockSpec((1, F, blk), lambda b, t: (b, 0, t)),
            pl.BlockSpec((H, F), lambda b, t: (0, 0)),
            pl.BlockSpec((H, 1), lambda b, t: (0, 0)),
            pl.BlockSpec((C, H), lambda b, t: (0, 0)),
            pl.BlockSpec((C, 1), lambda b, t: (0, 0)),
            pl.BlockSpec((G * FG, CG), lambda b, t: (0, 0)),
            pl.BlockSpec((1, G * blk, CG), lambda b, t: (b, t, 0)),
        ],
        out_specs=[
            pl.BlockSpec((1, F, blk), lambda b, t: (b, 0, t)),
            pl.BlockSpec((1, G, blk), lambda b, t: (b, 0, t)),
        ],
        out_shape=[
            jax.ShapeDtypeStruct((B, F, T), jnp.float32),
            jax.ShapeDtypeStruct((B, G, T), jnp.int32),
        ],
    )(series, W1, b1c, W2, b2c, cbT, gn3)
    return q, idx.transpose(0, 2, 1)


# speedup vs baseline: 2.0795x; 1.0326x over previous
import jax
import jax.numpy as jnp
from jax.experimental import pallas as pl


def _pick_block(t: int, target: int = 512) -> int:
    for cand in (target, 256, 128, 64, 32, 16, 8):
        if t % cand == 0 and cand <= t:
            return cand
    return t


def kernel(series, W1, b1, W2, b2, codebook, g_noise):
    B, F, T = series.shape
    H = W1.shape[0]
    C = W2.shape[0]
    G = F // codebook.shape[2]
    CG = C // G
    FG = F // G

    blk = _pick_block(T)

    nb = T // blk
    cbT = codebook.reshape(G, CG, FG).transpose(0, 2, 1).reshape(G * FG, CG)
    b1c = b1.reshape(H, 1)
    b2c = b2.reshape(C, 1)

    def kern(s_ref, w1_ref, b1_ref, w2_ref, b2_ref, cb_ref, gn_ref,
             q_ref, idx_ref):
        s = s_ref[0]
        h = jnp.dot(w1_ref[...], s, preferred_element_type=jnp.float32)
        h = jnp.maximum(h + b1_ref[...], 0.0)
        z = jnp.dot(w2_ref[...], h, preferred_element_type=jnp.float32)
        zb = z + b2_ref[...]
        gn_d = jnp.transpose(gn_ref[...].reshape(blk, G, CG), (1, 0, 2))
        idx_rows = []
        for g in range(G):
            l = zb[g * CG:(g + 1) * CG, :] + gn_d[g].T
            m = jnp.max(l, axis=0, keepdims=True)
            iota = jax.lax.broadcasted_iota(jnp.int32, (CG, blk), 0)
            idxg = jnp.min(jnp.where(l == m, iota, CG), axis=0, keepdims=True)
            oh = (iota == idxg).astype(jnp.float32)
            q_ref[0, g * FG:(g + 1) * FG, :] = jnp.dot(
                cb_ref[g * FG:(g + 1) * FG, :], oh,
                preferred_element_type=jnp.float32)
            idx_rows.append(idxg)
        idx_ref[0] = jnp.concatenate(idx_rows, axis=0)

    q, idx = pl.pallas_call(
        kern,
        grid=(B, T // blk),
        in_specs=[
            pl.BlockSpec((1, F, blk), lambda b, t: (b, 0, t)),
            pl.BlockSpec((H, F), lambda b, t: (0, 0)),
            pl.BlockSpec((H, 1), lambda b, t: (0, 0)),
            pl.BlockSpec((C, H), lambda b, t: (0, 0)),
            pl.BlockSpec((C, 1), lambda b, t: (0, 0)),
            pl.BlockSpec((G * FG, CG), lambda b, t: (0, 0)),
            pl.BlockSpec((G * blk, CG), lambda b, t, _nb=nb: (b * _nb + t, 0)),
        ],
        out_specs=[
            pl.BlockSpec((1, F, blk), lambda b, t: (b, 0, t)),
            pl.BlockSpec((1, G, blk), lambda b, t: (b, 0, t)),
        ],
        out_shape=[
            jax.ShapeDtypeStruct((B, F, T), jnp.float32),
            jax.ShapeDtypeStruct((B, G, T), jnp.int32),
        ],
    )(series, W1, b1c, W2, b2c, cbT, g_noise)
    return q, idx.transpose(0, 2, 1)


# speedup vs baseline: 2.9650x; 1.4258x over previous
import jax
import jax.numpy as jnp
from jax.experimental import pallas as pl


def _pick_block(t: int, target: int = 512) -> int:
    for cand in (target, 256, 128, 64, 32, 16, 8):
        if t % cand == 0 and cand <= t:
            return cand
    return t


def kernel(series, W1, b1, W2, b2, codebook, g_noise):
    B, F, T = series.shape
    H = W1.shape[0]
    C = W2.shape[0]
    G = F // codebook.shape[2]
    CG = C // G
    FG = F // G

    blk = _pick_block(T)

    gnT = g_noise.T
    nb = T // blk
    cbT = codebook.reshape(G, CG, FG).transpose(0, 2, 1).reshape(G * FG, CG)
    b1c = b1.reshape(H, 1)
    b2c = b2.reshape(C, 1)

    def kern(s_ref, w1_ref, b1_ref, w2_ref, b2_ref, cb_ref, gn_ref,
             q_ref, idx_ref):
        s = s_ref[0]
        h = jnp.dot(w1_ref[...], s, preferred_element_type=jnp.float32)
        h = jnp.maximum(h + b1_ref[...], 0.0)
        z = jnp.dot(w2_ref[...], h, preferred_element_type=jnp.float32)
        zb = z + b2_ref[...]
        gn_d = jnp.transpose(gn_ref[...].T.reshape(blk, G, CG), (1, 0, 2))
        idx_rows = []
        for g in range(G):
            l = zb[g * CG:(g + 1) * CG, :] + gn_d[g].T
            m = jnp.max(l, axis=0, keepdims=True)
            iota = jax.lax.broadcasted_iota(jnp.int32, (CG, blk), 0)
            idxg = jnp.min(jnp.where(l == m, iota, CG), axis=0, keepdims=True)
            oh = (iota == idxg).astype(jnp.float32)
            q_ref[0, g * FG:(g + 1) * FG, :] = jnp.dot(
                cb_ref[g * FG:(g + 1) * FG, :], oh,
                preferred_element_type=jnp.float32)
            idx_rows.append(idxg)
        idx_ref[0] = jnp.concatenate(idx_rows, axis=0)

    q, idx = pl.pallas_call(
        kern,
        grid=(B, T // blk),
        in_specs=[
            pl.BlockSpec((1, F, blk), lambda b, t: (b, 0, t)),
            pl.BlockSpec((H, F), lambda b, t: (0, 0)),
            pl.BlockSpec((H, 1), lambda b, t: (0, 0)),
            pl.BlockSpec((C, H), lambda b, t: (0, 0)),
            pl.BlockSpec((C, 1), lambda b, t: (0, 0)),
            pl.BlockSpec((G * FG, CG), lambda b, t: (0, 0)),
            pl.BlockSpec((CG, G * blk), lambda b, t, _nb=nb: (0, b * _nb + t)),
        ],
        out_specs=[
            pl.BlockSpec((1, F, blk), lambda b, t: (b, 0, t)),
            pl.BlockSpec((1, G, blk), lambda b, t: (b, 0, t)),
        ],
        out_shape=[
            jax.ShapeDtypeStruct((B, F, T), jnp.float32),
            jax.ShapeDtypeStruct((B, G, T), jnp.int32),
        ],
    )(series, W1, b1c, W2, b2c, cbT, gnT)
    return q, idx.transpose(0, 2, 1)


# speedup vs baseline: 3.3378x; 1.1257x over previous
import jax
import jax.numpy as jnp
from jax.experimental import pallas as pl


def _pick_block(t: int, target: int = 1024) -> int:
    for cand in (target, 512, 256, 128, 64, 32, 16, 8):
        if t % cand == 0 and cand <= t:
            return cand
    return t


def kernel(series, W1, b1, W2, b2, codebook, g_noise):
    B, F, T = series.shape
    H = W1.shape[0]
    C = W2.shape[0]
    G = F // codebook.shape[2]
    CG = C // G
    FG = F // G

    blk = _pick_block(T)

    gnT = g_noise.T
    nb = T // blk
    cbT = codebook.reshape(G, CG, FG).transpose(0, 2, 1).reshape(G * FG, CG)
    b1c = b1.reshape(H, 1)
    b2c = b2.reshape(C, 1)

    def kern(s_ref, w1_ref, b1_ref, w2_ref, b2_ref, cb_ref, gn_ref,
             q_ref, idx_ref):
        s = s_ref[0]
        h = jnp.dot(w1_ref[...], s, preferred_element_type=jnp.float32)
        h = jnp.maximum(h + b1_ref[...], 0.0)
        z = jnp.dot(w2_ref[...], h, preferred_element_type=jnp.float32)
        zb = z + b2_ref[...]
        gn_d = jnp.transpose(gn_ref[...].T.reshape(blk, G, CG), (1, 0, 2))
        idx_rows = []
        for g in range(G):
            l = zb[g * CG:(g + 1) * CG, :] + gn_d[g].T
            m = jnp.max(l, axis=0, keepdims=True)
            iota = jax.lax.broadcasted_iota(jnp.int32, (CG, blk), 0)
            idxg = jnp.min(jnp.where(l == m, iota, CG), axis=0, keepdims=True)
            oh = (iota == idxg).astype(jnp.float32)
            q_ref[0, g * FG:(g + 1) * FG, :] = jnp.dot(
                cb_ref[g * FG:(g + 1) * FG, :], oh,
                preferred_element_type=jnp.float32)
            idx_rows.append(idxg)
        idx_ref[0] = jnp.concatenate(idx_rows, axis=0)

    q, idx = pl.pallas_call(
        kern,
        grid=(B, T // blk),
        in_specs=[
            pl.BlockSpec((1, F, blk), lambda b, t: (b, 0, t)),
            pl.BlockSpec((H, F), lambda b, t: (0, 0)),
            pl.BlockSpec((H, 1), lambda b, t: (0, 0)),
            pl.BlockSpec((C, H), lambda b, t: (0, 0)),
            pl.BlockSpec((C, 1), lambda b, t: (0, 0)),
            pl.BlockSpec((G * FG, CG), lambda b, t: (0, 0)),
            pl.BlockSpec((CG, G * blk), lambda b, t, _nb=nb: (0, b * _nb + t)),
        ],
        out_specs=[
            pl.BlockSpec((1, F, blk), lambda b, t: (b, 0, t)),
            pl.BlockSpec((1, G, blk), lambda b, t: (b, 0, t)),
        ],
        out_shape=[
            jax.ShapeDtypeStruct((B, F, T), jnp.float32),
            jax.ShapeDtypeStruct((B, G, T), jnp.int32),
        ],
    )(series, W1, b1c, W2, b2c, cbT, gnT)
    return q, idx.transpose(0, 2, 1)
